# baseline (speedup 1.0000x reference)
import functools

import jax
import jax.numpy as jnp
from jax import lax
from jax.experimental import pallas as pl
from jax.experimental.pallas import tpu as pltpu
from jax.experimental.pallas import tpu_sc as plsc

NC = 2
NS = 16
L = 16
NW = NC * NS


@functools.cache
def _make_sampler(B, H, W, C, K):
    P = B * H * W
    assert P % (NW * K) == 0 and C % L == 0 and K % L == 0
    PPW = P // NW
    CHUNKS = PPW // K
    assert CHUNKS % 2 == 0

    mesh = plsc.VectorSubcoreMesh(core_axis_name="c", subcore_axis_name="s")

    def buf_set():
        return [
            pltpu.VMEM((K,), jnp.float32),
            pltpu.VMEM((K,), jnp.float32),
            pltpu.VMEM((4 * K,), jnp.int32),
            pltpu.VMEM((K + L,), jnp.float32),
            pltpu.VMEM((K + L,), jnp.float32),
            pltpu.VMEM((K + L,), jnp.float32),
            pltpu.VMEM((K + L,), jnp.float32),
            pltpu.VMEM((4 * K, C), jnp.float32),
            pltpu.VMEM((K * C // 128, 128), jnp.float32),
            pltpu.SemaphoreType.DMA,
            pltpu.SemaphoreType.DMA,
        ]

    @functools.partial(
        pl.kernel,
        mesh=mesh,
        compiler_params=pltpu.CompilerParams(use_tc_tiling_on_sc=False),
        out_type=jax.ShapeDtypeStruct((P * C // 128, 128), jnp.float32),
        scratch_types=[buf_set(), buf_set()],
    )
    def sampler(img_hbm, dx_hbm, dy_hbm, out_hbm, buf0, buf1):
        bufs = (buf0, buf1)
        wid = lax.axis_index("s") * NC + lax.axis_index("c")
        base = wid * PPW

        def fire(ci, b):
            (dxv, dyv, idx, wav, wbv, wcv, wdv, rows, _outv,
             gsem, _osem) = bufs[b]
            pix0 = pl.multiple_of(base + ci * K, 8)
            cdx = pltpu.async_copy(dx_hbm.at[pl.ds(pix0, K)], dxv, gsem)
            cdy = pltpu.async_copy(dy_hbm.at[pl.ds(pix0, K)], dyv, gsem)
            cdx.wait()
            cdy.wait()

            def grp(g, c2):
                sl = pl.ds(g * L, L)
                p = pix0 + g * L + lax.iota(jnp.int32, L)
                ww = lax.rem(p, W)
                hh = lax.rem(lax.div(p, W), H)
                bb = lax.div(p, W * H)
                x = dxv[sl] + ww.astype(jnp.float32) + 1.0
                y = dyv[sl] + hh.astype(jnp.float32) + 1.0
                xt = x.astype(jnp.int32)
                x0 = jnp.where(xt.astype(jnp.float32) > x, xt - 1, xt)
                yt = y.astype(jnp.int32)
                y0 = jnp.where(yt.astype(jnp.float32) > y, yt - 1, yt)
                x0c = jnp.clip(x0, 0, W + 1)
                x1c = jnp.clip(x0 + 1, 0, W + 1)
                y0c = jnp.clip(y0, 0, H + 1)
                y1c = jnp.clip(y0 + 1, 0, H + 1)
                ddx = x1c.astype(jnp.float32) - x
                ddy = y1c.astype(jnp.float32) - y
                wa = ddx * ddy
                wb = ddx * (1.0 - ddy)
                wc = (1.0 - ddx) * ddy
                wd = (1.0 - ddx) * (1.0 - ddy)
                rowb = bb * (H * W)

                def cidx(xi, yi):
                    col = jnp.clip(xi - 1, 0, W - 1)
                    row = jnp.clip(yi - 1, 0, H - 1)
                    return rowb + row * W + col

                def cw(xi, yi, wgt):
                    valid = (xi >= 1) & (xi <= W) & (yi >= 1) & (yi <= H)
                    return jnp.where(valid, wgt, 0.0)

                idx[pl.ds(g * L, L)] = cidx(x0c, y0c)
                wav[sl] = cw(x0c, y0c, wa)
                idx[pl.ds(K + g * L, L)] = cidx(x0c, y1c)
                wbv[sl] = cw(x0c, y1c, wb)
                idx[pl.ds(2 * K + g * L, L)] = cidx(x1c, y0c)
                wcv[sl] = cw(x1c, y0c, wc)
                idx[pl.ds(3 * K + g * L, L)] = cidx(x1c, y1c)
                wdv[sl] = cw(x1c, y1c, wd)
                return c2
            lax.fori_loop(0, K // L, grp, 0)
            pltpu.async_copy(img_hbm.at[idx], rows, gsem)

        def drain_combine(ci, b):
            (_dxv, _dyv, idx, wav, wbv, wcv, wdv, rows, outv,
             gsem, osem) = bufs[b]
            pix0 = pl.multiple_of(base + ci * K, 8)
            orow0 = pl.multiple_of(pix0 * C // 128, 8)
            OR = K * C // 128

            @pl.when(ci >= 2)
            def _():
                pltpu.make_async_copy(
                    outv, out_hbm.at[pl.ds(orow0, OR)], osem).wait()

            pltpu.make_async_copy(img_hbm.at[idx], rows, gsem).wait()

            def pix(pi, c2):
                was = wav[pl.ds(pi, L)][0]
                wbs = wbv[pl.ds(pi, L)][0]
                wcs = wcv[pl.ds(pi, L)][0]
                wds = wdv[pl.ds(pi, L)][0]
                f0 = pi * C
                for g in range(C // L):
                    s2 = pl.ds(g * L, L)
                    f = f0 + g * L
                    outv[f // 128, pl.ds(lax.rem(f, 128), L)] = (
                        was * rows[pi, s2]
                        + wbs * rows[K + pi, s2]
                        + wcs * rows[2 * K + pi, s2]
                        + wds * rows[3 * K + pi, s2])
                return c2
            lax.fori_loop(0, K, pix, 0)
            pltpu.async_copy(outv, out_hbm.at[pl.ds(orow0, OR)], osem)

        fire(0, 0)

        def outer(cio, carry):
            for s in range(2):
                ci = cio * 2 + s
                nci = ci + 1

                @pl.when(nci < CHUNKS)
                def _():
                    fire(nci, (s + 1) % 2)

                drain_combine(ci, s)
            return carry

        lax.fori_loop(0, CHUNKS // 2, outer, 0)
        for b in range(2):
            outv = bufs[b][8]
            osem = bufs[b][10]
            pltpu.make_async_copy(
                outv, out_hbm.at[pl.ds(base * C // 128, K * C // 128)],
                osem).wait()

    return sampler


def kernel(moving_image, deformation_matrix):
    B, H, W, C = moving_image.shape
    img_flat = moving_image.reshape(B * H * W, C)
    dx = deformation_matrix[..., 0].reshape(-1)
    dy = deformation_matrix[..., 1].reshape(-1)
    out = _make_sampler(B, H, W, C, 96)(img_flat, dx, dy)
    return out.reshape(B, H, W, C)


# speedup vs baseline: 1.4279x; 1.4279x over previous
import functools

import jax
import jax.numpy as jnp
from jax import lax
from jax.experimental import pallas as pl
from jax.experimental.pallas import tpu as pltpu
from jax.experimental.pallas import tpu_sc as plsc

NC = 2
NS = 16
L = 16
NW = NC * NS


@functools.cache
def _make_sampler(B, H, W, C, K):
    P = B * H * W
    assert P % (NW * K) == 0 and C % L == 0 and K % L == 0
    PPW = P // NW
    CHUNKS = PPW // K
    assert CHUNKS % 2 == 0

    mesh = plsc.VectorSubcoreMesh(core_axis_name="c", subcore_axis_name="s")

    def buf_set():
        return [
            pltpu.VMEM((K,), jnp.float32),
            pltpu.VMEM((K,), jnp.float32),
            pltpu.VMEM((4 * K,), jnp.int32),
            pltpu.VMEM((K + L,), jnp.float32),
            pltpu.VMEM((K + L,), jnp.float32),
            pltpu.VMEM((K + L,), jnp.float32),
            pltpu.VMEM((K + L,), jnp.float32),
            pltpu.VMEM((4 * K, C), jnp.float32),
            pltpu.VMEM((K, C), jnp.float32),
            pltpu.SemaphoreType.DMA,
            pltpu.SemaphoreType.DMA,
        ]

    @functools.partial(
        pl.kernel,
        mesh=mesh,
        compiler_params=pltpu.CompilerParams(use_tc_tiling_on_sc=False),
        out_type=jax.ShapeDtypeStruct((B, H, W, C), jnp.float32),
        scratch_types=[buf_set(), buf_set()],
    )
    def sampler(img_hbm, dx_hbm, dy_hbm, out4_hbm, buf0, buf1):
        bufs = (buf0, buf1)

        def out_slice(pix0):
            bb = lax.div(pix0, H * W)
            rem = lax.rem(pix0, H * W)
            hh = lax.div(rem, W)
            w0 = lax.rem(rem, W)
            return out4_hbm.at[bb, hh, pl.ds(w0, K)]
        wid = lax.axis_index("s") * NC + lax.axis_index("c")
        base = wid * PPW

        def fire(ci, b):
            (dxv, dyv, idx, wav, wbv, wcv, wdv, rows, _outv,
             gsem, _osem) = bufs[b]
            pix0 = pl.multiple_of(base + ci * K, 8)
            cdx = pltpu.async_copy(dx_hbm.at[pl.ds(pix0, K)], dxv, gsem)
            cdy = pltpu.async_copy(dy_hbm.at[pl.ds(pix0, K)], dyv, gsem)
            cdx.wait()
            cdy.wait()

            def grp(g, c2):
                sl = pl.ds(g * L, L)
                p = pix0 + g * L + lax.iota(jnp.int32, L)
                ww = lax.rem(p, W)
                hh = lax.rem(lax.div(p, W), H)
                bb = lax.div(p, W * H)
                x = dxv[sl] + ww.astype(jnp.float32) + 1.0
                y = dyv[sl] + hh.astype(jnp.float32) + 1.0
                xt = x.astype(jnp.int32)
                x0 = jnp.where(xt.astype(jnp.float32) > x, xt - 1, xt)
                yt = y.astype(jnp.int32)
                y0 = jnp.where(yt.astype(jnp.float32) > y, yt - 1, yt)
                x0c = jnp.clip(x0, 0, W + 1)
                x1c = jnp.clip(x0 + 1, 0, W + 1)
                y0c = jnp.clip(y0, 0, H + 1)
                y1c = jnp.clip(y0 + 1, 0, H + 1)
                ddx = x1c.astype(jnp.float32) - x
                ddy = y1c.astype(jnp.float32) - y
                wa = ddx * ddy
                wb = ddx * (1.0 - ddy)
                wc = (1.0 - ddx) * ddy
                wd = (1.0 - ddx) * (1.0 - ddy)
                rowb = bb * (H * W)

                def cidx(xi, yi):
                    col = jnp.clip(xi - 1, 0, W - 1)
                    row = jnp.clip(yi - 1, 0, H - 1)
                    return rowb + row * W + col

                def cw(xi, yi, wgt):
                    valid = (xi >= 1) & (xi <= W) & (yi >= 1) & (yi <= H)
                    return jnp.where(valid, wgt, 0.0)

                idx[pl.ds(g * L, L)] = cidx(x0c, y0c)
                wav[sl] = cw(x0c, y0c, wa)
                idx[pl.ds(K + g * L, L)] = cidx(x0c, y1c)
                wbv[sl] = cw(x0c, y1c, wb)
                idx[pl.ds(2 * K + g * L, L)] = cidx(x1c, y0c)
                wcv[sl] = cw(x1c, y0c, wc)
                idx[pl.ds(3 * K + g * L, L)] = cidx(x1c, y1c)
                wdv[sl] = cw(x1c, y1c, wd)
                return c2
            lax.fori_loop(0, K // L, grp, 0)
            pltpu.async_copy(img_hbm.at[idx], rows, gsem)

        def drain_combine(ci, b):
            (_dxv, _dyv, idx, wav, wbv, wcv, wdv, rows, outv,
             gsem, osem) = bufs[b]
            pix0 = pl.multiple_of(base + ci * K, 8)

            @pl.when(ci >= 2)
            def _():
                pltpu.make_async_copy(outv, out_slice(pix0), osem).wait()

            pltpu.make_async_copy(img_hbm.at[idx], rows, gsem).wait()

            def pix(pi, c2):
                was = wav[pl.ds(pi, L)][0]
                wbs = wbv[pl.ds(pi, L)][0]
                wcs = wcv[pl.ds(pi, L)][0]
                wds = wdv[pl.ds(pi, L)][0]
                for g in range(C // L):
                    s2 = pl.ds(g * L, L)
                    outv[pi, s2] = (
                        was * rows[pi, s2]
                        + wbs * rows[K + pi, s2]
                        + wcs * rows[2 * K + pi, s2]
                        + wds * rows[3 * K + pi, s2])
                return c2
            lax.fori_loop(0, K, pix, 0)
            pltpu.async_copy(outv, out_slice(pix0), osem)

        fire(0, 0)

        def outer(cio, carry):
            for s in range(2):
                ci = cio * 2 + s
                nci = ci + 1

                @pl.when(nci < CHUNKS)
                def _():
                    fire(nci, (s + 1) % 2)

                drain_combine(ci, s)
            return carry

        lax.fori_loop(0, CHUNKS // 2, outer, 0)
        for b in range(2):
            outv = bufs[b][8]
            osem = bufs[b][10]
            pltpu.make_async_copy(outv, out_slice(base), osem).wait()

    return sampler


def kernel(moving_image, deformation_matrix):
    B, H, W, C = moving_image.shape
    img_flat = moving_image.reshape(B * H * W, C)
    dx = deformation_matrix[..., 0].reshape(-1)
    dy = deformation_matrix[..., 1].reshape(-1)
    return _make_sampler(B, H, W, C, 96)(img_flat, dx, dy)


# speedup vs baseline: 1.4287x; 1.0006x over previous
import functools

import jax
import jax.numpy as jnp
from jax import lax
from jax.experimental import pallas as pl
from jax.experimental.pallas import tpu as pltpu
from jax.experimental.pallas import tpu_sc as plsc

NC = 2
NS = 16
L = 16
NW = NC * NS


@functools.cache
def _make_sampler(B, H, W, C, K):
    P = B * H * W
    assert P % (NW * K) == 0 and C % L == 0 and K % L == 0
    PPW = P // NW
    CHUNKS = PPW // K
    assert CHUNKS % 2 == 0

    mesh = plsc.VectorSubcoreMesh(core_axis_name="c", subcore_axis_name="s")

    def buf_set():
        return [
            pltpu.VMEM((K,), jnp.float32),
            pltpu.VMEM((K,), jnp.float32),
            pltpu.VMEM((4 * K,), jnp.int32),
            pltpu.VMEM((K + L,), jnp.float32),
            pltpu.VMEM((K + L,), jnp.float32),
            pltpu.VMEM((K + L,), jnp.float32),
            pltpu.VMEM((K + L,), jnp.float32),
            pltpu.VMEM((4 * K, C), jnp.float32),
            pltpu.VMEM((K, C), jnp.float32),
            pltpu.SemaphoreType.DMA,
            pltpu.SemaphoreType.DMA,
        ]

    @functools.partial(
        pl.kernel,
        mesh=mesh,
        compiler_params=pltpu.CompilerParams(use_tc_tiling_on_sc=False),
        out_type=jax.ShapeDtypeStruct((B, H, W, C), jnp.float32),
        scratch_types=[buf_set(), buf_set()],
    )
    def sampler(img_hbm, dx_hbm, dy_hbm, out4_hbm, buf0, buf1):
        bufs = (buf0, buf1)

        def out_slice(pix0):
            bb = lax.div(pix0, H * W)
            rem = lax.rem(pix0, H * W)
            hh = lax.div(rem, W)
            w0 = lax.rem(rem, W)
            return out4_hbm.at[bb, hh, pl.ds(w0, K)]
        wid = lax.axis_index("s") * NC + lax.axis_index("c")
        base = wid * PPW

        def fire(ci, b):
            (dxv, dyv, idx, wav, wbv, wcv, wdv, rows, _outv,
             gsem, _osem) = bufs[b]
            pix0 = pl.multiple_of(base + ci * K, 8)
            cdx = pltpu.async_copy(dx_hbm.at[pl.ds(pix0, K)], dxv, gsem)
            cdy = pltpu.async_copy(dy_hbm.at[pl.ds(pix0, K)], dyv, gsem)
            cdx.wait()
            cdy.wait()

            def grp(g, c2):
                sl = pl.ds(g * L, L)
                p = pix0 + g * L + lax.iota(jnp.int32, L)
                ww = lax.rem(p, W)
                hh = lax.rem(lax.div(p, W), H)
                bb = lax.div(p, W * H)
                x = dxv[sl] + ww.astype(jnp.float32) + 1.0
                y = dyv[sl] + hh.astype(jnp.float32) + 1.0
                xt = x.astype(jnp.int32)
                x0 = jnp.where(xt.astype(jnp.float32) > x, xt - 1, xt)
                yt = y.astype(jnp.int32)
                y0 = jnp.where(yt.astype(jnp.float32) > y, yt - 1, yt)
                x0c = jnp.clip(x0, 0, W + 1)
                x1c = jnp.clip(x0 + 1, 0, W + 1)
                y0c = jnp.clip(y0, 0, H + 1)
                y1c = jnp.clip(y0 + 1, 0, H + 1)
                ddx = x1c.astype(jnp.float32) - x
                ddy = y1c.astype(jnp.float32) - y
                wa = ddx * ddy
                wb = ddx * (1.0 - ddy)
                wc = (1.0 - ddx) * ddy
                wd = (1.0 - ddx) * (1.0 - ddy)
                rowb = bb * (H * W)

                def cidx(xi, yi):
                    col = jnp.clip(xi - 1, 0, W - 1)
                    row = jnp.clip(yi - 1, 0, H - 1)
                    return rowb + row * W + col

                def cw(xi, yi, wgt):
                    valid = (xi >= 1) & (xi <= W) & (yi >= 1) & (yi <= H)
                    return jnp.where(valid, wgt, 0.0)

                idx[pl.ds(g * L, L)] = cidx(x0c, y0c)
                wav[sl] = cw(x0c, y0c, wa)
                idx[pl.ds(K + g * L, L)] = cidx(x0c, y1c)
                wbv[sl] = cw(x0c, y1c, wb)
                idx[pl.ds(2 * K + g * L, L)] = cidx(x1c, y0c)
                wcv[sl] = cw(x1c, y0c, wc)
                idx[pl.ds(3 * K + g * L, L)] = cidx(x1c, y1c)
                wdv[sl] = cw(x1c, y1c, wd)
                return c2
            lax.fori_loop(0, K // L, grp, 0)
            pltpu.async_copy(img_hbm.at[idx], rows, gsem)

        def drain_combine(ci, b):
            (_dxv, _dyv, idx, wav, wbv, wcv, wdv, rows, outv,
             gsem, osem) = bufs[b]
            pix0 = pl.multiple_of(base + ci * K, 8)

            @pl.when(ci >= 2)
            def _():
                pltpu.make_async_copy(outv, out_slice(pix0), osem).wait()

            pltpu.make_async_copy(img_hbm.at[idx], rows, gsem).wait()

            def pix(pi, c2):
                was = wav[pl.ds(pi, L)][0]
                wbs = wbv[pl.ds(pi, L)][0]
                wcs = wcv[pl.ds(pi, L)][0]
                wds = wdv[pl.ds(pi, L)][0]
                for g in range(C // L):
                    s2 = pl.ds(g * L, L)
                    outv[pi, s2] = (
                        was * rows[pi, s2]
                        + wbs * rows[K + pi, s2]
                        + wcs * rows[2 * K + pi, s2]
                        + wds * rows[3 * K + pi, s2])
                return c2
            lax.fori_loop(0, K, pix, 0)
            pltpu.async_copy(outv, out_slice(pix0), osem)

        fire(0, 0)

        def outer(cio, carry):
            for s in range(2):
                ci = cio * 2 + s
                nci = ci + 1

                @pl.when(nci < CHUNKS)
                def _():
                    fire(nci, (s + 1) % 2)

                drain_combine(ci, s)
            return carry

        lax.fori_loop(0, CHUNKS // 2, outer, 0)
        for b in range(2):
            outv = bufs[b][8]
            osem = bufs[b][10]
            pltpu.make_async_copy(outv, out_slice(base), osem).wait()

    return sampler


def kernel(moving_image, deformation_matrix):
    B, H, W, C = moving_image.shape
    img_flat = moving_image.reshape(B * H * W, C)
    img_flat = jax.lax.optimization_barrier(img_flat)
    dx = deformation_matrix[..., 0].reshape(-1)
    dy = deformation_matrix[..., 1].reshape(-1)
    return _make_sampler(B, H, W, C, 96)(img_flat, dx, dy)


# speedup vs baseline: 1.4921x; 1.0444x over previous
import functools

import jax
import jax.numpy as jnp
from jax import lax
from jax.experimental import pallas as pl
from jax.experimental.pallas import tpu as pltpu
from jax.experimental.pallas import tpu_sc as plsc

NC = 2
NS = 16
L = 16
NW = NC * NS


@functools.cache
def _make_sampler(B, H, W, C, K):
    P = B * H * W
    assert P % (NW * K) == 0 and C % L == 0 and K % L == 0
    PPW = P // NW
    CHUNKS = PPW // K
    assert CHUNKS % 2 == 0
    assert (H * W) % PPW == 0 and W % K == 0

    mesh = plsc.VectorSubcoreMesh(core_axis_name="c", subcore_axis_name="s")

    def buf_set():
        return [
            pltpu.VMEM((K,), jnp.float32),
            pltpu.VMEM((K,), jnp.float32),
            pltpu.VMEM((4 * K,), jnp.int32),
            pltpu.VMEM((K + L,), jnp.float32),
            pltpu.VMEM((K + L,), jnp.float32),
            pltpu.VMEM((K + L,), jnp.float32),
            pltpu.VMEM((K + L,), jnp.float32),
            pltpu.VMEM((4 * K, C), jnp.float32),
            pltpu.VMEM((K, C), jnp.float32),
            pltpu.SemaphoreType.DMA,
            pltpu.SemaphoreType.DMA,
        ]

    @functools.partial(
        pl.kernel,
        mesh=mesh,
        compiler_params=pltpu.CompilerParams(use_tc_tiling_on_sc=False),
        out_type=jax.ShapeDtypeStruct((B, H, W, C), jnp.float32),
        scratch_types=[buf_set(), buf_set()],
    )
    def sampler(img3_hbm, dx_hbm, dy_hbm, out4_hbm, buf0, buf1):
        bufs = (buf0, buf1)

        def out_slice(pix0):
            bb = lax.div(pix0, H * W)
            rem = lax.rem(pix0, H * W)
            hh = lax.div(rem, W)
            w0 = lax.rem(rem, W)
            return out4_hbm.at[bb, hh, pl.ds(w0, K)]
        wid = lax.axis_index("s") * NC + lax.axis_index("c")
        base = wid * PPW
        img_hbm = img3_hbm.at[lax.div(base, H * W)]

        def fire(ci, b):
            (dxv, dyv, idx, wav, wbv, wcv, wdv, rows, _outv,
             gsem, _osem) = bufs[b]
            pix0 = pl.multiple_of(base + ci * K, 8)
            cdx = pltpu.async_copy(dx_hbm.at[pl.ds(pix0, K)], dxv, gsem)
            cdy = pltpu.async_copy(dy_hbm.at[pl.ds(pix0, K)], dyv, gsem)
            cdx.wait()
            cdy.wait()

            def grp(g, c2):
                sl = pl.ds(g * L, L)
                p = pix0 + g * L + lax.iota(jnp.int32, L)
                ww = lax.rem(p, W)
                hh = lax.rem(lax.div(p, W), H)
                x = dxv[sl] + ww.astype(jnp.float32) + 1.0
                y = dyv[sl] + hh.astype(jnp.float32) + 1.0
                xt = x.astype(jnp.int32)
                x0 = jnp.where(xt.astype(jnp.float32) > x, xt - 1, xt)
                yt = y.astype(jnp.int32)
                y0 = jnp.where(yt.astype(jnp.float32) > y, yt - 1, yt)
                x0c = jnp.clip(x0, 0, W + 1)
                x1c = jnp.clip(x0 + 1, 0, W + 1)
                y0c = jnp.clip(y0, 0, H + 1)
                y1c = jnp.clip(y0 + 1, 0, H + 1)
                ddx = x1c.astype(jnp.float32) - x
                ddy = y1c.astype(jnp.float32) - y
                wa = ddx * ddy
                wb = ddx * (1.0 - ddy)
                wc = (1.0 - ddx) * ddy
                wd = (1.0 - ddx) * (1.0 - ddy)

                def cidx(xi, yi):
                    col = jnp.clip(xi - 1, 0, W - 1)
                    row = jnp.clip(yi - 1, 0, H - 1)
                    return row * W + col

                def cw(xi, yi, wgt):
                    valid = (xi >= 1) & (xi <= W) & (yi >= 1) & (yi <= H)
                    return jnp.where(valid, wgt, 0.0)

                idx[pl.ds(g * L, L)] = cidx(x0c, y0c)
                wav[sl] = cw(x0c, y0c, wa)
                idx[pl.ds(K + g * L, L)] = cidx(x0c, y1c)
                wbv[sl] = cw(x0c, y1c, wb)
                idx[pl.ds(2 * K + g * L, L)] = cidx(x1c, y0c)
                wcv[sl] = cw(x1c, y0c, wc)
                idx[pl.ds(3 * K + g * L, L)] = cidx(x1c, y1c)
                wdv[sl] = cw(x1c, y1c, wd)
                return c2
            lax.fori_loop(0, K // L, grp, 0)
            pltpu.async_copy(img_hbm.at[idx], rows, gsem)

        def drain_combine(ci, b):
            (_dxv, _dyv, idx, wav, wbv, wcv, wdv, rows, outv,
             gsem, osem) = bufs[b]
            pix0 = pl.multiple_of(base + ci * K, 8)

            @pl.when(ci >= 2)
            def _():
                pltpu.make_async_copy(outv, out_slice(pix0), osem).wait()

            pltpu.make_async_copy(img_hbm.at[idx], rows, gsem).wait()

            def pix(pi, c2):
                was = wav[pl.ds(pi, L)][0]
                wbs = wbv[pl.ds(pi, L)][0]
                wcs = wcv[pl.ds(pi, L)][0]
                wds = wdv[pl.ds(pi, L)][0]
                for g in range(C // L):
                    s2 = pl.ds(g * L, L)
                    outv[pi, s2] = (
                        was * rows[pi, s2]
                        + wbs * rows[K + pi, s2]
                        + wcs * rows[2 * K + pi, s2]
                        + wds * rows[3 * K + pi, s2])
                return c2
            lax.fori_loop(0, K, pix, 0)
            pltpu.async_copy(outv, out_slice(pix0), osem)

        fire(0, 0)

        def outer(cio, carry):
            for s in range(2):
                ci = cio * 2 + s
                nci = ci + 1

                @pl.when(nci < CHUNKS)
                def _():
                    fire(nci, (s + 1) % 2)

                drain_combine(ci, s)
            return carry

        lax.fori_loop(0, CHUNKS // 2, outer, 0)
        for b in range(2):
            outv = bufs[b][8]
            osem = bufs[b][10]
            pltpu.make_async_copy(outv, out_slice(base), osem).wait()

    return sampler


def kernel(moving_image, deformation_matrix):
    B, H, W, C = moving_image.shape
    img3 = moving_image.reshape(B, H * W, C)
    dx = deformation_matrix[..., 0].reshape(-1)
    dy = deformation_matrix[..., 1].reshape(-1)
    return _make_sampler(B, H, W, C, 96)(img3, dx, dy)
